# baseline (speedup 1.0000x reference)
import functools

import jax
import jax.numpy as jnp
from jax import lax
from jax.experimental import pallas as pl
from jax.experimental.pallas import tpu as pltpu
from jax.experimental.pallas import tpu_sc as plsc

VOCAB = 100000
EMB_DIM = 64
BATCH = 4096
HIST = 50

_NC, _NS = 2, 16
_NW = _NC * _NS
_B_PER_W = BATCH // _NW
_IDX_PER_W = _B_PER_W * HIST
_GROUPS = _B_PER_W // 16

_SUBCH = 2
_SUBROWS = EMB_DIM // _SUBCH


def _tc_matvec_body(embt_hbm, w_ref, p_ref, *scratch):
    bufs, sems = scratch[:_SUBCH], scratch[_SUBCH:]

    for i in range(_SUBCH):
        pltpu.make_async_copy(
            embt_hbm.at[pl.ds(i * _SUBROWS, _SUBROWS), :], bufs[i], sems[i]
        ).start()

    acc = None
    for i in range(_SUBCH):
        pltpu.make_async_copy(
            embt_hbm.at[pl.ds(i * _SUBROWS, _SUBROWS), :], bufs[i], sems[i]
        ).wait()
        part = lax.dot_general(
            w_ref[pl.ds(0, 1), pl.ds(i * _SUBROWS, _SUBROWS)], bufs[i][...],
            dimension_numbers=(((1,), (0,)), ((), ())),
            preferred_element_type=jnp.float32,
        )
        acc = part if acc is None else acc + part
    p_ref[...] = acc[0]


_tc_matvec = pl.pallas_call(
    _tc_matvec_body,
    in_specs=[
        pl.BlockSpec(memory_space=pl.ANY),
        pl.BlockSpec((1, EMB_DIM), lambda: (0, 0)),
    ],
    out_specs=pl.BlockSpec((VOCAB,), lambda: (0,)),
    out_shape=jax.ShapeDtypeStruct((VOCAB,), jnp.float32),
    scratch_shapes=(
        [pltpu.VMEM((_SUBROWS, VOCAB), jnp.float32)] * _SUBCH
        + [pltpu.SemaphoreType.DMA] * _SUBCH
    ),
)


def _sc_body(p_hbm, xt_hbm, mask_hbm, b_hbm, out_hbm, p_v, xt_v, mask_v, b_v, out_v, psem):
    wid = lax.axis_index("s") * _NC + lax.axis_index("c")
    pcopy = pltpu.make_async_copy(p_hbm, p_v, psem)
    pcopy.start()
    pltpu.sync_copy(xt_hbm.at[:, pl.ds(wid * _B_PER_W, _B_PER_W)], xt_v)
    pltpu.sync_copy(mask_hbm.at[0, pl.ds(wid * _B_PER_W, _B_PER_W)], mask_v)
    pltpu.sync_copy(b_hbm, b_v)
    pcopy.wait()

    def body(t, accs):
        new = []
        for g in range(_GROUPS):
            xi = xt_v[t, pl.ds(g * 16, 16)]
            new.append(accs[g] + plsc.load_gather(p_v, [xi]))
        return tuple(new)

    zero = jnp.zeros((16,), jnp.float32)
    accs = lax.fori_loop(
        0, HIST, body, tuple(zero for _ in range(_GROUPS)), unroll=5
    )

    for g in range(_GROUPS):
        val = accs[g] / mask_v[pl.ds(g * 16, 16)] + b_v[...]
        out_v[pl.ds(g * 16, 16)] = 1.0 / (1.0 + jnp.exp(-val))

    pltpu.sync_copy(out_v, out_hbm.at[pl.ds(wid * _B_PER_W, _B_PER_W)])


@functools.cache
def _sc_pool():
    return pl.kernel(
        _sc_body,
        out_type=jax.ShapeDtypeStruct((BATCH,), jnp.float32),
        mesh=plsc.VectorSubcoreMesh(
            core_axis_name="c", subcore_axis_name="s", num_cores=_NC, num_subcores=_NS
        ),
        compiler_params=pltpu.CompilerParams(needs_layout_passes=False),
        scratch_types=[
            pltpu.VMEM((VOCAB,), jnp.float32),
            pltpu.VMEM((HIST, _B_PER_W), jnp.int32),
            pltpu.VMEM((_B_PER_W,), jnp.float32),
            pltpu.VMEM((16,), jnp.float32),
            pltpu.VMEM((_B_PER_W,), jnp.float32),
            pltpu.SemaphoreType.DMA,
        ],
    )


def kernel(X, X_mask, emb, W, b):
    p = _tc_matvec(emb.T, W)
    b16 = jnp.broadcast_to(b.astype(jnp.float32), (16,))
    return _sc_pool()(p, X.T, X_mask.T, b16)


# speedup vs baseline: 1.0244x; 1.0244x over previous
import functools

import jax
import jax.numpy as jnp
from jax import lax
from jax.experimental import pallas as pl
from jax.experimental.pallas import tpu as pltpu
from jax.experimental.pallas import tpu_sc as plsc

VOCAB = 100000
EMB_DIM = 64
BATCH = 4096
HIST = 50

_NC, _NS = 2, 16
_NW = _NC * _NS
_B_PER_W = BATCH // _NW
_IDX_PER_W = _B_PER_W * HIST
_GROUPS = _B_PER_W // 16

_SUBCH = 4
_SUBROWS = EMB_DIM // _SUBCH


def _tc_matvec_body(embt_hbm, w_ref, p_ref, *scratch):
    bufs, sems = scratch[:_SUBCH], scratch[_SUBCH:]

    for i in range(_SUBCH):
        pltpu.make_async_copy(
            embt_hbm.at[pl.ds(i * _SUBROWS, _SUBROWS), :], bufs[i], sems[i]
        ).start()

    acc = None
    for i in range(_SUBCH):
        pltpu.make_async_copy(
            embt_hbm.at[pl.ds(i * _SUBROWS, _SUBROWS), :], bufs[i], sems[i]
        ).wait()
        part = lax.dot_general(
            w_ref[pl.ds(0, 1), pl.ds(i * _SUBROWS, _SUBROWS)], bufs[i][...],
            dimension_numbers=(((1,), (0,)), ((), ())),
            preferred_element_type=jnp.float32,
        )
        acc = part if acc is None else acc + part
    p_ref[...] = acc[0]


_tc_matvec = pl.pallas_call(
    _tc_matvec_body,
    in_specs=[
        pl.BlockSpec(memory_space=pl.ANY),
        pl.BlockSpec((1, EMB_DIM), lambda: (0, 0)),
    ],
    out_specs=pl.BlockSpec((VOCAB,), lambda: (0,)),
    out_shape=jax.ShapeDtypeStruct((VOCAB,), jnp.float32),
    scratch_shapes=(
        [pltpu.VMEM((_SUBROWS, VOCAB), jnp.float32)] * _SUBCH
        + [pltpu.SemaphoreType.DMA] * _SUBCH
    ),
)


def _sc_body(p_hbm, xt_hbm, mask_hbm, b_hbm, out_hbm, p_v, xt_v, mask_v, b_v, out_v, psem):
    wid = lax.axis_index("s") * _NC + lax.axis_index("c")
    pcopy = pltpu.make_async_copy(p_hbm, p_v, psem)
    pcopy.start()
    pltpu.sync_copy(xt_hbm.at[:, pl.ds(wid * _B_PER_W, _B_PER_W)], xt_v)
    pltpu.sync_copy(mask_hbm.at[0, pl.ds(wid * _B_PER_W, _B_PER_W)], mask_v)
    pltpu.sync_copy(b_hbm, b_v)
    pcopy.wait()

    def body(t, accs):
        new = []
        for g in range(_GROUPS):
            xi = xt_v[t, pl.ds(g * 16, 16)]
            new.append(accs[g] + plsc.load_gather(p_v, [xi]))
        return tuple(new)

    zero = jnp.zeros((16,), jnp.float32)
    accs = lax.fori_loop(
        0, HIST, body, tuple(zero for _ in range(_GROUPS)), unroll=5
    )

    for g in range(_GROUPS):
        val = accs[g] / mask_v[pl.ds(g * 16, 16)] + b_v[...]
        out_v[pl.ds(g * 16, 16)] = 1.0 / (1.0 + jnp.exp(-val))

    pltpu.sync_copy(out_v, out_hbm.at[pl.ds(wid * _B_PER_W, _B_PER_W)])


@functools.cache
def _sc_pool():
    return pl.kernel(
        _sc_body,
        out_type=jax.ShapeDtypeStruct((BATCH,), jnp.float32),
        mesh=plsc.VectorSubcoreMesh(
            core_axis_name="c", subcore_axis_name="s", num_cores=_NC, num_subcores=_NS
        ),
        compiler_params=pltpu.CompilerParams(needs_layout_passes=False),
        scratch_types=[
            pltpu.VMEM((VOCAB,), jnp.float32),
            pltpu.VMEM((HIST, _B_PER_W), jnp.int32),
            pltpu.VMEM((_B_PER_W,), jnp.float32),
            pltpu.VMEM((16,), jnp.float32),
            pltpu.VMEM((_B_PER_W,), jnp.float32),
            pltpu.SemaphoreType.DMA,
        ],
    )


def kernel(X, X_mask, emb, W, b):
    p = _tc_matvec(emb.T, W)
    b16 = jnp.broadcast_to(b.astype(jnp.float32), (16,))
    return _sc_pool()(p, X.T, X_mask.T, b16)
